# baseline (speedup 1.0000x reference)
import functools

import jax
import jax.numpy as jnp
from jax import lax
from jax.experimental import pallas as pl
from jax.experimental.pallas import tpu as pltpu
from jax.experimental.pallas import tpu_sc as plsc

N_NODES = 10000
N_EDGES = 320000
D_FEAT = 128
N_REL = 8

NC = 1
NS = 16
NW = NC * NS
EPT = N_EDGES // NW
BATCH = 80
HALF = 40
NBATCH = EPT // BATCH
NB2 = 2
G = 5
NG = NBATCH // G
N_PAD = 10240
ROWS_PT = N_PAD // NS
FL = 624


def _tc_transform(feat, weight):
    def body(feat_ref, w_ref, out_ref):
        out_ref[...] = jnp.dot(feat_ref[...], w_ref[0],
                               preferred_element_type=jnp.float32)

    return pl.pallas_call(
        body,
        grid=(N_REL,),
        in_specs=[
            pl.BlockSpec((N_NODES, D_FEAT), lambda r: (0, 0)),
            pl.BlockSpec((1, D_FEAT, D_FEAT), lambda r: (r, 0, 0)),
        ],
        out_specs=pl.BlockSpec((N_NODES, D_FEAT), lambda r: (r, 0)),
        out_shape=jax.ShapeDtypeStruct((N_REL * N_NODES, D_FEAT),
                                       jnp.float32),
    )(feat, weight)


def _tc_index_prep(src2, et2):
    eb = N_EDGES // 128

    def body(src_ref, et_ref, out_ref):
        out_ref[...] = et_ref[...] * N_NODES + src_ref[...]

    return pl.pallas_call(
        body,
        in_specs=[
            pl.BlockSpec((eb, 128), lambda: (0, 0)),
            pl.BlockSpec((eb, 128), lambda: (0, 0)),
        ],
        out_specs=pl.BlockSpec((eb, 128), lambda: (0, 0)),
        out_shape=jax.ShapeDtypeStruct((eb, 128), jnp.int32),
    )(src2, et2)


def _sc_gather_scatter(table, idx4, dst4, zeros):
    mesh = plsc.VectorSubcoreMesh(core_axis_name="c", subcore_axis_name="s",
                                  num_cores=NC)

    @functools.partial(
        pl.kernel,
        out_type=jax.ShapeDtypeStruct((N_NODES, D_FEAT), jnp.float32),
        mesh=mesh,
        scratch_types=[
            pltpu.VMEM((3, G, BATCH), jnp.int32),
            pltpu.VMEM((3, G, BATCH), jnp.int32),
            pltpu.VMEM((NB2, BATCH, D_FEAT), jnp.float32),
            pltpu.VMEM_SHARED((N_PAD, D_FEAT), jnp.float32),
            pltpu.SemaphoreType.DMA((NB2,)),
            pltpu.SemaphoreType.DMA,
        ],
    )
    def body(table_hbm, idx_hbm, dst_hbm, z_hbm, out_hbm,
             idxb, dstb, rows_v, acc, gsems, bsem):
        s = lax.axis_index("s")

        def gather_batch(nblk, q, b):
            for h in range(2):
                pltpu.async_copy(
                    table_hbm.at[idxb.at[nblk, q, pl.ds(h * HALF, HALF)]],
                    rows_v.at[b, pl.ds(h * HALF, HALF)], gsems.at[b])

        pltpu.sync_copy(idx_hbm.at[s, 0], idxb.at[0])
        pltpu.sync_copy(dst_hbm.at[s, 0], dstb.at[0])
        for b in range(NB2):
            gather_batch(0, b, b)
        pltpu.async_copy(idx_hbm.at[s, 1], idxb.at[1], bsem)
        pltpu.async_copy(dst_hbm.at[s, 1], dstb.at[1], bsem)
        pltpu.sync_copy(z_hbm, acc.at[pl.ds(s * ROWS_PT, ROWS_PT)])
        plsc.subcore_barrier()

        def group(g, carry):
            @pl.when(g + 1 < NG)
            def _():
                pltpu.make_async_copy(idx_hbm.at[s, g + 1],
                                      idxb.at[(g + 1) % 3], bsem).wait()
                pltpu.make_async_copy(dst_hbm.at[s, g + 1],
                                      dstb.at[(g + 1) % 3], bsem).wait()

            @pl.when(g + 2 < NG)
            def _():
                pltpu.async_copy(idx_hbm.at[s, g + 2],
                                 idxb.at[(g + 2) % 3], bsem)
                pltpu.async_copy(dst_hbm.at[s, g + 2],
                                 dstb.at[(g + 2) % 3], bsem)

            blk = g % 3
            for q in range(G):
                j = g * G + q
                b = q % NB2
                pltpu.make_async_copy(table_hbm.at[idxb.at[blk, q]],
                                      rows_v.at[b], gsems.at[b]).wait()
                pltpu.sync_copy(rows_v.at[b], acc.at[dstb.at[blk, q]],
                                add=True)

                @pl.when(j + NB2 < NBATCH)
                def _():
                    qq = q + NB2
                    nblk = lax.select(qq < G, blk, (g + 1) % 3)
                    gather_batch(nblk, qq % G, b)
            return carry

        lax.fori_loop(0, NG, group, 0)
        plsc.subcore_barrier()
        pltpu.sync_copy(acc.at[pl.ds(s * FL, FL)],
                        out_hbm.at[pl.ds(s * FL, FL)])

        @pl.when(s == NS - 1)
        def _():
            pltpu.sync_copy(acc.at[pl.ds(NS * FL, N_NODES - NS * FL)],
                            out_hbm.at[pl.ds(NS * FL, N_NODES - NS * FL)])

    return body(table, idx4, dst4, zeros)


def kernel(feat, edge_index, etypes, weight):
    table = _tc_transform(feat, weight)
    src2 = edge_index[0].reshape(-1, 128)
    et2 = etypes.reshape(-1, 128)
    idx4 = _tc_index_prep(src2, et2).reshape(NS, NG, G, BATCH)
    dst4 = edge_index[1].reshape(NS, NG, G, BATCH)
    zeros = jnp.zeros((ROWS_PT, D_FEAT), jnp.float32)
    return _sc_gather_scatter(table, idx4, dst4, zeros)


# speedup vs baseline: 1.3112x; 1.3112x over previous
import functools

import jax
import jax.numpy as jnp
from jax import lax
from jax.experimental import pallas as pl
from jax.experimental.pallas import tpu as pltpu
from jax.experimental.pallas import tpu_sc as plsc

N_NODES = 10000
N_EDGES = 320000
D_FEAT = 128
N_REL = 8

NC = 1
NS = 16
NW = NC * NS
EPT = N_EDGES // NW
CHUNK = 32
NCHUNK = EPT // CHUNK
NBUF = 5
G = 5
NG = NCHUNK // G
N_PAD = 10240
ROWS_PT = N_PAD // NS
FL = 624


def _tc_transform(feat, weight):
    def body(feat_ref, w_ref, out_ref):
        out_ref[...] = jnp.dot(feat_ref[...], w_ref[0],
                               preferred_element_type=jnp.float32)

    return pl.pallas_call(
        body,
        grid=(N_REL,),
        in_specs=[
            pl.BlockSpec((N_NODES, D_FEAT), lambda r: (0, 0)),
            pl.BlockSpec((1, D_FEAT, D_FEAT), lambda r: (r, 0, 0)),
        ],
        out_specs=pl.BlockSpec((N_NODES, D_FEAT), lambda r: (r, 0)),
        out_shape=jax.ShapeDtypeStruct((N_REL * N_NODES, D_FEAT),
                                       jnp.float32),
    )(feat, weight)


def _tc_index_prep(src2, et2):
    eb = N_EDGES // 128

    def body(src_ref, et_ref, out_ref):
        out_ref[...] = et_ref[...] * N_NODES + src_ref[...]

    return pl.pallas_call(
        body,
        in_specs=[
            pl.BlockSpec((eb, 128), lambda: (0, 0)),
            pl.BlockSpec((eb, 128), lambda: (0, 0)),
        ],
        out_specs=pl.BlockSpec((eb, 128), lambda: (0, 0)),
        out_shape=jax.ShapeDtypeStruct((eb, 128), jnp.int32),
    )(src2, et2)


def _sc_gather_scatter(table, idx4, dst4, zeros):
    mesh = plsc.VectorSubcoreMesh(core_axis_name="c", subcore_axis_name="s",
                                  num_cores=NC)

    @functools.partial(
        pl.kernel,
        out_type=jax.ShapeDtypeStruct((N_NODES, D_FEAT), jnp.float32),
        mesh=mesh,
        scratch_types=[
            pltpu.VMEM((4, G, CHUNK), jnp.int32),
            pltpu.VMEM((4, G, CHUNK), jnp.int32),
            pltpu.VMEM((NBUF, CHUNK, D_FEAT), jnp.float32),
            pltpu.VMEM_SHARED((N_PAD, D_FEAT), jnp.float32),
            pltpu.SemaphoreType.DMA((NBUF,)),
            pltpu.SemaphoreType.DMA,
        ],
    )
    def body(table_hbm, idx_hbm, dst_hbm, z_hbm, out_hbm,
             idxb, dstb, rows_v, acc, gsems, bsem):
        s = lax.axis_index("s")
        pltpu.sync_copy(idx_hbm.at[s, 0], idxb.at[0])
        pltpu.sync_copy(dst_hbm.at[s, 0], dstb.at[0])
        for b in range(NBUF):
            pltpu.async_copy(table_hbm.at[idxb.at[0, b]], rows_v.at[b],
                             gsems.at[b])
        pltpu.async_copy(idx_hbm.at[s, 1], idxb.at[1], bsem)
        pltpu.async_copy(dst_hbm.at[s, 1], dstb.at[1], bsem)
        pltpu.sync_copy(z_hbm, acc.at[pl.ds(s * ROWS_PT, ROWS_PT)])
        plsc.subcore_barrier()

        def group(g, carry):
            @pl.when(g + 1 < NG)
            def _():
                pltpu.make_async_copy(idx_hbm.at[s, g + 1],
                                      idxb.at[(g + 1) % 4], bsem).wait()
                pltpu.make_async_copy(dst_hbm.at[s, g + 1],
                                      dstb.at[(g + 1) % 4], bsem).wait()

            @pl.when(g + 2 < NG)
            def _():
                pltpu.async_copy(idx_hbm.at[s, g + 2],
                                 idxb.at[(g + 2) % 4], bsem)
                pltpu.async_copy(dst_hbm.at[s, g + 2],
                                 dstb.at[(g + 2) % 4], bsem)

            blk = g % 4
            for k in range(G):
                j = g * G + k
                b = k % NBUF
                pltpu.make_async_copy(table_hbm.at[idxb.at[blk, k]],
                                      rows_v.at[b], gsems.at[b]).wait()
                pltpu.sync_copy(rows_v.at[b], acc.at[dstb.at[blk, k]],
                                add=True)

                @pl.when(j + NBUF < NCHUNK)
                def _():
                    kk = k + NBUF
                    nblk = lax.select(kk < G, blk, (g + 1) % 4)
                    pltpu.async_copy(
                        table_hbm.at[idxb.at[nblk, kk % G]],
                        rows_v.at[b], gsems.at[b])
            return carry

        lax.fori_loop(0, NG, group, 0)
        plsc.subcore_barrier()
        pltpu.sync_copy(acc.at[pl.ds(s * FL, FL)],
                        out_hbm.at[pl.ds(s * FL, FL)])

        @pl.when(s == NS - 1)
        def _():
            pltpu.sync_copy(acc.at[pl.ds(NS * FL, N_NODES - NS * FL)],
                            out_hbm.at[pl.ds(NS * FL, N_NODES - NS * FL)])

    return body(table, idx4, dst4, zeros)


def kernel(feat, edge_index, etypes, weight):
    table = _tc_transform(feat, weight)
    src2 = edge_index[0].reshape(-1, 128)
    et2 = etypes.reshape(-1, 128)
    idx4 = _tc_index_prep(src2, et2).reshape(NS, NG, G, CHUNK)
    dst4 = edge_index[1].reshape(NS, NG, G, CHUNK)
    zeros = jnp.zeros((ROWS_PT, D_FEAT), jnp.float32)
    return _sc_gather_scatter(table, idx4, dst4, zeros)
